# baseline (speedup 1.0000x reference)
import functools

import jax
import jax.numpy as jnp
from jax import lax
from jax.experimental import pallas as pl
from jax.experimental.pallas import tpu as pltpu
from jax.experimental.pallas import tpu_sc as plsc

N = 8388608
NC = 2
NS = 16
L = 16
NW = NC * NS
PER_W = N // NW
CHUNK = 16384
NCHUNK = PER_W // CHUNK
SLICES = CHUNK // L

_mesh = plsc.VectorSubcoreMesh(core_axis_name="c", subcore_axis_name="s")


@functools.partial(
    pl.kernel,
    out_type=jax.ShapeDtypeStruct((NW, 4, L), jnp.float32),
    mesh=_mesh,
    scratch_types=[
        pltpu.VMEM((2, CHUNK), jnp.float32),
        pltpu.VMEM((2, CHUNK), jnp.int32),
        pltpu.VMEM((2, CHUNK), jnp.int32),
        pltpu.VMEM((4, L), jnp.float32),
        pltpu.SemaphoreType.DMA,
        pltpu.SemaphoreType.DMA,
    ],
)
def _partial_sums(yp_hbm, s_hbm, g_hbm, out_hbm,
                  yp_buf, s_buf, g_buf, res_v, sem0, sem1):
    wid = lax.axis_index("s") * NC + lax.axis_index("c")
    base = wid * PER_W
    sems = (sem0, sem1)

    def start(c, slot):
        off = base + c * CHUNK
        return (
            pltpu.async_copy(yp_hbm.at[pl.ds(off, CHUNK)], yp_buf.at[slot],
                             sems[slot]),
            pltpu.async_copy(s_hbm.at[pl.ds(off, CHUNK)], s_buf.at[slot],
                             sems[slot]),
            pltpu.async_copy(g_hbm.at[pl.ds(off, CHUNK)], g_buf.at[slot],
                             sems[slot]),
        )

    def chunk_body(slot, accs):
        @plsc.parallel_loop(0, CHUNK, step=L, unroll=8, carry=accs)
        def body(o, accs):
            acc_sv, acc_cv, acc_s1, acc_c1 = accs
            yp = yp_buf[slot, pl.ds(o, L)]
            sv = s_buf[slot, pl.ds(o, L)]
            gv = g_buf[slot, pl.ds(o, L)]
            m1 = gv & sv
            gf = gv.astype(jnp.float32)
            m1f = m1.astype(jnp.float32)
            return (acc_sv + yp * gf, acc_cv + gf,
                    acc_s1 + yp * m1f, acc_c1 + m1f)
        return body

    zero = jnp.zeros((L,), jnp.float32)
    accs = (zero, zero, zero, zero)
    inflight = [None, None]
    inflight[0] = start(0, 0)
    for c in range(NCHUNK):
        if c + 1 < NCHUNK:
            inflight[(c + 1) % 2] = start(c + 1, (c + 1) % 2)
        for cp in inflight[c % 2]:
            cp.wait()
        accs = chunk_body(c % 2, accs)

    acc_sv, acc_cv, acc_s1, acc_c1 = accs
    res_v[0, :] = acc_sv - acc_s1
    res_v[1, :] = acc_cv - acc_c1
    res_v[2, :] = acc_s1
    res_v[3, :] = acc_c1
    pltpu.sync_copy(res_v, out_hbm.at[wid])


def kernel(y_pred, s, y_gt):
    y_pred = y_pred.reshape(-1)
    s = s.reshape(-1).astype(jnp.int32)
    y_gt = y_gt.reshape(-1).astype(jnp.int32)
    p = _partial_sums(y_pred, s, y_gt)
    t = jnp.sum(p, axis=(0, 2))
    return jnp.abs(t[0] / t[1] - t[2] / t[3])


# speedup vs baseline: 1.6501x; 1.6501x over previous
import functools

import jax
import jax.numpy as jnp
from jax import lax
from jax.experimental import pallas as pl
from jax.experimental.pallas import tpu as pltpu
from jax.experimental.pallas import tpu_sc as plsc

N = 8388608
NC = 2
NS = 16
L = 16
NW = NC * NS
PER_W = N // NW
CHUNK = 16384
NCHUNK = PER_W // CHUNK
GRP = 8 * L
NGRP = CHUNK // GRP

_mesh = plsc.VectorSubcoreMesh(core_axis_name="c", subcore_axis_name="s")


@functools.partial(
    pl.kernel,
    out_type=jax.ShapeDtypeStruct((NW, 4, L), jnp.float32),
    mesh=_mesh,
    scratch_types=[
        pltpu.VMEM((2, CHUNK), jnp.float32),
        pltpu.VMEM((2, CHUNK), jnp.int32),
        pltpu.VMEM((2, CHUNK), jnp.int32),
        pltpu.VMEM((4, L), jnp.float32),
        pltpu.SemaphoreType.DMA,
        pltpu.SemaphoreType.DMA,
    ],
)
def _partial_sums(yp_hbm, s_hbm, g_hbm, out_hbm,
                  yp_buf, s_buf, g_buf, res_v, sem0, sem1):
    wid = lax.axis_index("s") * NC + lax.axis_index("c")
    base = wid * PER_W
    sems = (sem0, sem1)

    def drain(slot):
        pltpu.make_async_copy(yp_hbm.at[pl.ds(0, CHUNK)], yp_buf.at[slot],
                              sems[slot]).wait()
        pltpu.make_async_copy(s_hbm.at[pl.ds(0, CHUNK)], s_buf.at[slot],
                              sems[slot]).wait()
        pltpu.make_async_copy(g_hbm.at[pl.ds(0, CHUNK)], g_buf.at[slot],
                              sems[slot]).wait()

    off0 = base
    pltpu.async_copy(yp_hbm.at[pl.ds(off0, CHUNK)], yp_buf.at[0], sem0)
    pltpu.async_copy(s_hbm.at[pl.ds(off0, CHUNK)], s_buf.at[0], sem0)
    pltpu.async_copy(g_hbm.at[pl.ds(off0, CHUNK)], g_buf.at[0], sem0)

    zero = jnp.zeros((L,), jnp.float32)
    accs = (zero, zero, zero, zero)
    for c in range(NCHUNK):
        slot, nslot = c % 2, (c + 1) % 2
        prefetch = c + 1 < NCHUNK
        noff = base + (c + 1) * CHUNK
        drain(slot)

        @plsc.parallel_loop(0, CHUNK, step=GRP, unroll=2, carry=accs)
        def body(o, accs, slot=slot, nslot=nslot, prefetch=prefetch,
                 noff=noff):
            if prefetch:
                pltpu.async_copy(yp_hbm.at[pl.ds(noff + o, GRP)],
                                 yp_buf.at[nslot, pl.ds(o, GRP)], sems[nslot])
                pltpu.async_copy(s_hbm.at[pl.ds(noff + o, GRP)],
                                 s_buf.at[nslot, pl.ds(o, GRP)], sems[nslot])
                pltpu.async_copy(g_hbm.at[pl.ds(noff + o, GRP)],
                                 g_buf.at[nslot, pl.ds(o, GRP)], sems[nslot])
            acc_sv, acc_cv, acc_s1, acc_c1 = accs
            tsv, tcv, ts1, tc1 = [], [], [], []
            for j in range(GRP // L):
                yp = yp_buf[slot, pl.ds(o + j * L, L)]
                sv = s_buf[slot, pl.ds(o + j * L, L)]
                gv = g_buf[slot, pl.ds(o + j * L, L)]
                m1 = gv & sv
                gf = gv.astype(jnp.float32)
                m1f = m1.astype(jnp.float32)
                tsv.append(yp * gf)
                tcv.append(gf)
                ts1.append(yp * m1f)
                tc1.append(m1f)

            def tree(xs):
                while len(xs) > 1:
                    xs = [a + b for a, b in zip(xs[::2], xs[1::2])]
                return xs[0]

            return (acc_sv + tree(tsv), acc_cv + tree(tcv),
                    acc_s1 + tree(ts1), acc_c1 + tree(tc1))

        accs = body

    acc_sv, acc_cv, acc_s1, acc_c1 = accs
    res_v[0, :] = acc_sv - acc_s1
    res_v[1, :] = acc_cv - acc_c1
    res_v[2, :] = acc_s1
    res_v[3, :] = acc_c1
    pltpu.sync_copy(res_v, out_hbm.at[wid])


def kernel(y_pred, s, y_gt):
    y_pred = y_pred.reshape(-1)
    s = s.reshape(-1).astype(jnp.int32)
    y_gt = y_gt.reshape(-1).astype(jnp.int32)
    p = _partial_sums(y_pred, s, y_gt)
    t = jnp.sum(p, axis=(0, 2))
    return jnp.abs(t[0] / t[1] - t[2] / t[3])


# speedup vs baseline: 1.8100x; 1.0969x over previous
import functools

import jax
import jax.numpy as jnp
from jax import lax
from jax.experimental import pallas as pl
from jax.experimental.pallas import tpu as pltpu
from jax.experimental.pallas import tpu_sc as plsc

N = 8388608
NC = 2
NS = 16
L = 16
NW = NC * NS
CHUNK = 16384
GRP = 8 * L

TC_PART = 12
TC_ELEMS = (N // 16) * TC_PART
SC_BASE = TC_ELEMS
PER_W = (N - TC_ELEMS) // NW
NCHUNK = PER_W // CHUNK

LANES = 128
TC_ROWS = TC_ELEMS // LANES
BR = 1024

_mesh = plsc.VectorSubcoreMesh(core_axis_name="c", subcore_axis_name="s")


@functools.partial(
    pl.kernel,
    out_type=jax.ShapeDtypeStruct((NW, 4, L), jnp.float32),
    mesh=_mesh,
    scratch_types=[
        pltpu.VMEM((2, CHUNK), jnp.float32),
        pltpu.VMEM((2, CHUNK), jnp.int32),
        pltpu.VMEM((2, CHUNK), jnp.int32),
        pltpu.VMEM((4, L), jnp.float32),
        pltpu.SemaphoreType.DMA,
        pltpu.SemaphoreType.DMA,
    ],
)
def _sc_partial_sums(yp_hbm, s_hbm, g_hbm, out_hbm,
                     yp_buf, s_buf, g_buf, res_v, sem0, sem1):
    wid = lax.axis_index("s") * NC + lax.axis_index("c")
    base = SC_BASE + wid * PER_W
    sems = (sem0, sem1)

    def drain(slot):
        pltpu.make_async_copy(yp_hbm.at[pl.ds(0, CHUNK)], yp_buf.at[slot],
                              sems[slot]).wait()
        pltpu.make_async_copy(s_hbm.at[pl.ds(0, CHUNK)], s_buf.at[slot],
                              sems[slot]).wait()
        pltpu.make_async_copy(g_hbm.at[pl.ds(0, CHUNK)], g_buf.at[slot],
                              sems[slot]).wait()

    pltpu.async_copy(yp_hbm.at[pl.ds(base, CHUNK)], yp_buf.at[0], sem0)
    pltpu.async_copy(s_hbm.at[pl.ds(base, CHUNK)], s_buf.at[0], sem0)
    pltpu.async_copy(g_hbm.at[pl.ds(base, CHUNK)], g_buf.at[0], sem0)

    zero = jnp.zeros((L,), jnp.float32)
    accs = (zero, zero, zero, zero)
    for c in range(NCHUNK):
        slot, nslot = c % 2, (c + 1) % 2
        prefetch = c + 1 < NCHUNK
        noff = base + (c + 1) * CHUNK
        drain(slot)

        @plsc.parallel_loop(0, CHUNK, step=GRP, unroll=2, carry=accs)
        def body(o, accs, slot=slot, nslot=nslot, prefetch=prefetch,
                 noff=noff):
            if prefetch:
                pltpu.async_copy(yp_hbm.at[pl.ds(noff + o, GRP)],
                                 yp_buf.at[nslot, pl.ds(o, GRP)], sems[nslot])
                pltpu.async_copy(s_hbm.at[pl.ds(noff + o, GRP)],
                                 s_buf.at[nslot, pl.ds(o, GRP)], sems[nslot])
                pltpu.async_copy(g_hbm.at[pl.ds(noff + o, GRP)],
                                 g_buf.at[nslot, pl.ds(o, GRP)], sems[nslot])
            acc_sv, acc_cv, acc_s1, acc_c1 = accs
            tsv, tcv, ts1, tc1 = [], [], [], []
            for j in range(GRP // L):
                yp = yp_buf[slot, pl.ds(o + j * L, L)]
                sv = s_buf[slot, pl.ds(o + j * L, L)]
                gv = g_buf[slot, pl.ds(o + j * L, L)]
                m1 = gv & sv
                gf = gv.astype(jnp.float32)
                m1f = m1.astype(jnp.float32)
                tsv.append(yp * gf)
                tcv.append(gf)
                ts1.append(yp * m1f)
                tc1.append(m1f)

            def tree(xs):
                while len(xs) > 1:
                    xs = [a + b for a, b in zip(xs[::2], xs[1::2])]
                return xs[0]

            return (acc_sv + tree(tsv), acc_cv + tree(tcv),
                    acc_s1 + tree(ts1), acc_c1 + tree(tc1))

        accs = body

    acc_sv, acc_cv, acc_s1, acc_c1 = accs
    res_v[0, :] = acc_sv - acc_s1
    res_v[1, :] = acc_cv - acc_c1
    res_v[2, :] = acc_s1
    res_v[3, :] = acc_c1
    pltpu.sync_copy(res_v, out_hbm.at[wid])


def _tc_body(yp_ref, s_ref, g_ref, out_ref):
    i = pl.program_id(0)

    @pl.when(i == 0)
    def _():
        out_ref[...] = jnp.zeros_like(out_ref)

    yp = yp_ref[...]
    sv = s_ref[...]
    gv = g_ref[...]
    m1 = gv & sv
    gf = gv.astype(jnp.float32)
    m1f = m1.astype(jnp.float32)
    upd = jnp.concatenate([
        jnp.sum(yp * gf, axis=0, keepdims=True),
        jnp.sum(gf, axis=0, keepdims=True),
        jnp.sum(yp * m1f, axis=0, keepdims=True),
        jnp.sum(m1f, axis=0, keepdims=True),
    ], axis=0)
    out_ref[...] += upd


_tc_partial_sums = pl.pallas_call(
    _tc_body,
    grid=(TC_ROWS // BR,),
    in_specs=[
        pl.BlockSpec((BR, LANES), lambda i: (i, 0)),
        pl.BlockSpec((BR, LANES), lambda i: (i, 0)),
        pl.BlockSpec((BR, LANES), lambda i: (i, 0)),
    ],
    out_specs=pl.BlockSpec((4, LANES), lambda i: (0, 0)),
    out_shape=jax.ShapeDtypeStruct((4, LANES), jnp.float32),
    compiler_params=pltpu.CompilerParams(
        dimension_semantics=("arbitrary",),
    ),
)


def kernel(y_pred, s, y_gt):
    y_pred = y_pred.reshape(-1)
    s = s.reshape(-1).astype(jnp.int32)
    y_gt = y_gt.reshape(-1).astype(jnp.int32)
    p_sc = _sc_partial_sums(y_pred, s, y_gt)
    p_tc = _tc_partial_sums(
        y_pred.reshape(N // LANES, LANES),
        s.reshape(N // LANES, LANES),
        y_gt.reshape(N // LANES, LANES),
    )
    t = jnp.sum(p_sc, axis=(0, 2)) + jnp.sum(p_tc, axis=1)
    return jnp.abs(t[0] / t[1] - t[2] / t[3])
